# baseline (speedup 1.0000x reference)
import functools

import jax
import jax.numpy as jnp
from jax import lax
from jax.experimental import pallas as pl
from jax.experimental.pallas import tpu as pltpu
from jax.experimental.pallas import tpu_sc as plsc


def _route_weights(topk_idx, topk_probs, e):
    t = topk_idx.shape[0]
    info = plsc.get_sparse_core_info()
    nw = info.num_cores * info.num_subcores
    nl = info.num_lanes
    tpw = t // nw
    mesh = plsc.VectorSubcoreMesh(core_axis_name="c", subcore_axis_name="s")

    @functools.partial(
        pl.kernel, mesh=mesh,
        out_type=jax.ShapeDtypeStruct((e, t), jnp.float32),
        scratch_types=[
            pltpu.VMEM((tpw,), jnp.int32),
            pltpu.VMEM((tpw,), jnp.int32),
            pltpu.VMEM((tpw,), jnp.float32),
            pltpu.VMEM((tpw,), jnp.float32),
            pltpu.VMEM((e, tpw), jnp.float32),
        ],
    )
    def route(idx0_hbm, idx1_hbm, p0_hbm, p1_hbm, out_hbm,
              idx0_v, idx1_v, p0_v, p1_v, acc_v):
        wid = lax.axis_index("s") * info.num_cores + lax.axis_index("c")
        base = wid * tpw
        pltpu.sync_copy(idx0_hbm.at[pl.ds(base, tpw)], idx0_v)
        pltpu.sync_copy(idx1_hbm.at[pl.ds(base, tpw)], idx1_v)
        pltpu.sync_copy(p0_hbm.at[pl.ds(base, tpw)], p0_v)
        pltpu.sync_copy(p1_hbm.at[pl.ds(base, tpw)], p1_v)
        for s in range(tpw // nl):
            i0 = idx0_v[pl.ds(s * nl, nl)]
            i1 = idx1_v[pl.ds(s * nl, nl)]
            p0 = p0_v[pl.ds(s * nl, nl)]
            p1 = p1_v[pl.ds(s * nl, nl)]
            for ex in range(e):
                exv = jnp.full((nl,), ex, jnp.int32)
                zf = jnp.zeros((nl,), jnp.float32)
                val = (jnp.where(i0 == exv, p0, zf)
                       + jnp.where(i1 == exv, p1, zf))
                acc_v[ex, pl.ds(s * nl, nl)] = val
        for ex in range(e):
            pltpu.sync_copy(acc_v.at[ex],
                            out_hbm.at[ex, pl.ds(base, tpw)])

    idx0 = topk_idx[:, 0].astype(jnp.int32)
    idx1 = topk_idx[:, 1].astype(jnp.int32)
    p0 = topk_probs[:, 0].astype(jnp.float32)
    p1 = topk_probs[:, 1].astype(jnp.float32)
    return route(idx0, idx1, p0, p1)


def _gelu_exact(v):
    return 0.5 * v * (1.0 + jax.lax.erf(v * 0.7071067811865476))


def _base_body(x_ref, W1_ref, b1_ref, W2_ref, b2_ref, o_ref, *, kh, hid):
    f32 = jnp.float32
    x = x_ref[...]
    acc = jnp.broadcast_to(b2_ref[...], o_ref.shape).astype(f32)
    for k in range(hid // kh):
        h = _gelu_exact(
            jnp.dot(x, W1_ref[:, k * kh:(k + 1) * kh],
                    preferred_element_type=f32) + b1_ref[:, k * kh:(k + 1) * kh])
        acc = acc + jnp.dot(h, W2_ref[k * kh:(k + 1) * kh, :],
                            preferred_element_type=f32)
    o_ref[...] = acc


def _lora_body(x_ref, base_ref, wt_ref, exp_ref, Wd_ref, Wu_ref, o_ref):
    f32 = jnp.float32
    x = x_ref[...]
    w = jax.lax.dot_general(wt_ref[...], exp_ref[...],
                            (((0,), (0,)), ((), ())),
                            preferred_element_type=f32)
    h2 = _gelu_exact(jnp.dot(x, Wd_ref[...], preferred_element_type=f32))
    o_ref[...] = base_ref[...] + jnp.dot(h2 * w, Wu_ref[...],
                                         preferred_element_type=f32)


def kernel(x, gate, topk_probs, topk_idx, W1, b1, W2, b2, w_down, w_up):
    del gate
    orig_shape = x.shape
    dim = x.shape[-1]
    e, _, r = w_down.shape
    hid = W1.shape[1]
    alpha = 8.0
    scaling = alpha / r

    xf = x.reshape(-1, dim)
    t = xf.shape[0]
    tm = min(1024, t)
    kh = min(512, hid)
    lanes = 128

    wt = _route_weights(topk_idx, topk_probs, e)

    eidx = jnp.arange(lanes) // r
    exp = (jnp.arange(e)[:, None] == eidx[None, :]).astype(jnp.float32)

    wd = jnp.transpose(w_down, (1, 0, 2)).reshape(dim, e * r)
    wd = jnp.pad(wd, ((0, 0), (0, lanes - e * r)))
    wu = w_up.reshape(e * r, dim) * scaling
    wu = jnp.pad(wu, ((0, lanes - e * r), (0, 0)))

    b1r = b1.reshape(1, hid)
    b2r = b2.reshape(1, dim)

    grid = (t // tm,)
    base_out = pl.pallas_call(
        functools.partial(_base_body, kh=kh, hid=hid),
        grid=grid,
        in_specs=[
            pl.BlockSpec((tm, dim), lambda i: (i, 0)),
            pl.BlockSpec((dim, hid), lambda i: (0, 0)),
            pl.BlockSpec((1, hid), lambda i: (0, 0)),
            pl.BlockSpec((hid, dim), lambda i: (0, 0)),
            pl.BlockSpec((1, dim), lambda i: (0, 0)),
        ],
        out_specs=pl.BlockSpec((tm, dim), lambda i: (i, 0)),
        out_shape=jax.ShapeDtypeStruct((t, dim), jnp.float32),
    )(xf, W1, b1r, W2, b2r)

    out = pl.pallas_call(
        _lora_body,
        grid=grid,
        in_specs=[
            pl.BlockSpec((tm, dim), lambda i: (i, 0)),
            pl.BlockSpec((tm, dim), lambda i: (i, 0)),
            pl.BlockSpec((e, tm), lambda i: (0, i)),
            pl.BlockSpec((e, lanes), lambda i: (0, 0)),
            pl.BlockSpec((dim, lanes), lambda i: (0, 0)),
            pl.BlockSpec((lanes, dim), lambda i: (0, 0)),
        ],
        out_specs=pl.BlockSpec((tm, dim), lambda i: (i, 0)),
        out_shape=jax.ShapeDtypeStruct((t, dim), jnp.float32),
    )(xf, base_out, wt, exp, wd, wu)
    return out.reshape(orig_shape)


# speedup vs baseline: 1.0928x; 1.0928x over previous
import functools

import jax
import jax.numpy as jnp
from jax import lax
from jax.experimental import pallas as pl
from jax.experimental.pallas import tpu as pltpu
from jax.experimental.pallas import tpu_sc as plsc


def _route_weights(topk_idx, topk_probs, e):
    t = topk_idx.shape[0]
    info = plsc.get_sparse_core_info()
    nw = info.num_cores * info.num_subcores
    nl = info.num_lanes
    tpw = t // nw
    mesh = plsc.VectorSubcoreMesh(core_axis_name="c", subcore_axis_name="s")

    @functools.partial(
        pl.kernel, mesh=mesh,
        out_type=jax.ShapeDtypeStruct((e, t), jnp.float32),
        scratch_types=[
            pltpu.VMEM((4, tpw), jnp.float32),
            pltpu.VMEM((e, tpw), jnp.float32),
        ],
    )
    def route(pack_hbm, out_hbm, pack_v, acc_v):
        wid = lax.axis_index("s") * info.num_cores + lax.axis_index("c")
        base = wid * tpw
        pltpu.sync_copy(pack_hbm.at[:, pl.ds(base, tpw)], pack_v)
        for s in range(tpw // nl):
            p0 = pack_v[0, pl.ds(s * nl, nl)]
            p1 = pack_v[1, pl.ds(s * nl, nl)]
            i0 = pack_v[2, pl.ds(s * nl, nl)]
            i1 = pack_v[3, pl.ds(s * nl, nl)]
            for ex in range(e):
                exv = jnp.full((nl,), float(ex), jnp.float32)
                zf = jnp.zeros((nl,), jnp.float32)
                val = (jnp.where(i0 == exv, p0, zf)
                       + jnp.where(i1 == exv, p1, zf))
                acc_v[ex, pl.ds(s * nl, nl)] = val
        for ex in range(e):
            pltpu.sync_copy(acc_v.at[ex],
                            out_hbm.at[ex, pl.ds(base, tpw)])

    pack = jnp.concatenate([topk_probs.astype(jnp.float32).T,
                            topk_idx.astype(jnp.float32).T], axis=0)
    return route(pack)


def _gelu_exact(v):
    return 0.5 * v * (1.0 + jax.lax.erf(v * 0.7071067811865476))


def _fused_body(x_ref, wt_ref, exp_ref, W1_ref, b1_ref, W2_ref, b2_ref,
                Wd_ref, Wu_ref, o_ref, *, kh, hid):
    f32 = jnp.float32
    x = x_ref[...]

    w = jax.lax.dot_general(wt_ref[...], exp_ref[...],
                            (((0,), (0,)), ((), ())),
                            preferred_element_type=f32)

    h2 = _gelu_exact(jnp.dot(x, Wd_ref[...], preferred_element_type=f32))
    acc = jnp.dot(h2 * w, Wu_ref[...], preferred_element_type=f32)
    acc = acc + b2_ref[...]

    for k in range(hid // kh):
        h = _gelu_exact(
            jnp.dot(x, W1_ref[:, k * kh:(k + 1) * kh],
                    preferred_element_type=f32) + b1_ref[:, k * kh:(k + 1) * kh])
        acc = acc + jnp.dot(h, W2_ref[k * kh:(k + 1) * kh, :],
                            preferred_element_type=f32)
    o_ref[...] = acc


def kernel(x, gate, topk_probs, topk_idx, W1, b1, W2, b2, w_down, w_up):
    del gate
    orig_shape = x.shape
    dim = x.shape[-1]
    e, _, r = w_down.shape
    hid = W1.shape[1]
    alpha = 8.0
    scaling = alpha / r

    xf = x.reshape(-1, dim)
    t = xf.shape[0]
    tm = min(1024, t)
    kh = min(512, hid)
    lanes = 128

    wt = _route_weights(topk_idx, topk_probs, e)

    eidx = jnp.arange(lanes) // r
    exp = (jnp.arange(e)[:, None] == eidx[None, :]).astype(jnp.float32)

    wd = jnp.transpose(w_down, (1, 0, 2)).reshape(dim, e * r)
    wd = jnp.pad(wd, ((0, 0), (0, lanes - e * r)))
    wu = w_up.reshape(e * r, dim) * scaling
    wu = jnp.pad(wu, ((0, lanes - e * r), (0, 0)))

    b1r = b1.reshape(1, hid)
    b2r = b2.reshape(1, dim)

    grid = (t // tm,)
    out = pl.pallas_call(
        functools.partial(_fused_body, kh=kh, hid=hid),
        grid=grid,
        in_specs=[
            pl.BlockSpec((tm, dim), lambda i: (i, 0)),
            pl.BlockSpec((e, tm), lambda i: (0, i)),
            pl.BlockSpec((e, lanes), lambda i: (0, 0)),
            pl.BlockSpec((dim, hid), lambda i: (0, 0)),
            pl.BlockSpec((1, hid), lambda i: (0, 0)),
            pl.BlockSpec((hid, dim), lambda i: (0, 0)),
            pl.BlockSpec((1, dim), lambda i: (0, 0)),
            pl.BlockSpec((dim, lanes), lambda i: (0, 0)),
            pl.BlockSpec((lanes, dim), lambda i: (0, 0)),
        ],
        out_specs=pl.BlockSpec((tm, dim), lambda i: (i, 0)),
        out_shape=jax.ShapeDtypeStruct((t, dim), jnp.float32),
    )(xf, wt, exp, W1, b1r, W2, b2r, wd, wu)
    return out.reshape(orig_shape)
